# Initial kernel scaffold; baseline (speedup 1.0000x reference)
import jax
import jax.numpy as jnp
from jax.experimental import pallas as pl


def kernel(x_time, W_month, W_day, W_weekday, W_hour):
    raise NotImplementedError("write your pallas kernel here")


# speedup vs baseline: 15.2572x; 15.2572x over previous
import jax
import jax.numpy as jnp
from jax import lax
from jax.experimental import pallas as pl
from jax.experimental.pallas import tpu as pltpu
from jax.experimental.pallas import tpu_sc as plsc

D_MODEL = 32
BATCH = 4096
SEQ = 200
TOTAL = BATCH * SEQ
NC, NS, L = 2, 16, 16
NW = NC * NS
ROWS_PER_W = TOTAL // NW
CHUNK = 256
NCHUNK = ROWS_PER_W // CHUNK
IDX_ROWS = CHUNK // 128


def _build_table_body(wm_ref, wd_ref, ww_ref, wh_ref, t_ref):
    i = lax.broadcasted_iota(jnp.int32, (4096, 1), 0)
    parts = []
    for shift, rows, w_ref in ((9, 8, wm_ref), (6, 8, wd_ref),
                               (3, 7, ww_ref), (0, 8, wh_ref)):
        sub = (i >> shift) & 7
        k = lax.broadcasted_iota(jnp.int32, (1, rows), 1)
        onehot = (sub == k).astype(jnp.float32)
        parts.append(jnp.dot(onehot, w_ref[pl.ds(0, rows), :],
                             preferred_element_type=jnp.float32))
    t_ref[...] = jnp.concatenate(parts, axis=1)


def _gather_body(xt_hbm, t_hbm, out_hbm, idx_v, cidx_v, rows_v, sem):
    wid = lax.axis_index("s") * NC + lax.axis_index("c")
    base = wid * ROWS_PER_W

    def chunk_body(c, carry):
        off = base + c * CHUNK

        for f in range(4):
            for j in range(IDX_ROWS):
                pltpu.sync_copy(
                    xt_hbm.at[f, pl.ds(off + j * 128, 128)],
                    idx_v.at[f * IDX_ROWS + j])

        for j in range(IDX_ROWS):
            for p in range(0, 128, L):
                s = pl.ds(p, L)
                i0 = idx_v[0 * IDX_ROWS + j, s]
                i1 = idx_v[1 * IDX_ROWS + j, s]
                i2 = idx_v[2 * IDX_ROWS + j, s]
                i3 = idx_v[3 * IDX_ROWS + j, s]
                cidx_v[j, s] = (i0 << 9) | (i1 << 6) | (i2 << 3) | i3

        copies = []
        for j in range(IDX_ROWS):
            copies.append(pltpu.async_copy(
                t_hbm.at[cidx_v.at[j]],
                rows_v.at[pl.ds(j * 128, 128)],
                sem))
        for cp in copies:
            cp.wait()
        pltpu.sync_copy(rows_v, out_hbm.at[pl.ds(off, CHUNK)])
        return carry

    lax.fori_loop(0, NCHUNK, chunk_body, 0)


@jax.jit
def _run(xt_t, wm, wd, ww, wh):
    table = pl.pallas_call(
        _build_table_body,
        out_shape=jax.ShapeDtypeStruct((4096, 4 * D_MODEL), jnp.float32),
    )(wm, wd, ww, wh)

    mesh = plsc.VectorSubcoreMesh(
        core_axis_name="c", subcore_axis_name="s",
        num_cores=NC, num_subcores=NS)
    gather = pl.kernel(
        _gather_body,
        out_type=jax.ShapeDtypeStruct((TOTAL, 4 * D_MODEL), jnp.float32),
        mesh=mesh,
        scratch_types=[
            pltpu.VMEM((4 * IDX_ROWS, 128), jnp.int32),
            pltpu.VMEM((IDX_ROWS, 128), jnp.int32),
            pltpu.VMEM((CHUNK, 4 * D_MODEL), jnp.float32),
            pltpu.SemaphoreType.DMA,
        ],
    )
    return gather(xt_t, table)


def kernel(x_time, W_month, W_day, W_weekday, W_hour):
    xt_t = x_time.astype(jnp.int32).reshape(TOTAL, 4).T
    out = _run(xt_t, W_month, W_day, W_weekday, W_hour)
    return out.reshape(BATCH, SEQ, 4 * D_MODEL)


# speedup vs baseline: 26.1532x; 1.7142x over previous
import jax
import jax.numpy as jnp
from jax import lax
from jax.experimental import pallas as pl
from jax.experimental.pallas import tpu as pltpu
from jax.experimental.pallas import tpu_sc as plsc

D_MODEL = 32
D_OUT = 4 * D_MODEL
BATCH = 4096
SEQ = 200
TOTAL = BATCH * SEQ
NC, NS = 2, 16
NW = NC * NS
ROWS_PER_W = TOTAL // NW
CHUNK = 256
NCHUNK = ROWS_PER_W // CHUNK
IDX_ROWS = CHUNK // 128
IDX_TOTAL = TOTAL // 128
IDX_PER_W = IDX_TOTAL // NW


def _build_table_body(wm_ref, wd_ref, ww_ref, wh_ref, t_ref):
    i = lax.broadcasted_iota(jnp.int32, (4096, 1), 0)
    parts = []
    for shift, rows, w_ref in ((9, 8, wm_ref), (6, 8, wd_ref),
                               (3, 7, ww_ref), (0, 8, wh_ref)):
        sub = (i >> shift) & 7
        acc = jnp.broadcast_to(w_ref[0:1, :], (4096, D_MODEL))
        for k in range(1, rows):
            acc = jnp.where(sub == k, w_ref[k:k + 1, :], acc)
        parts.append(acc)
    t_ref[...] = jnp.concatenate(parts, axis=1)


def _fuse_idx_body(x0_ref, x1_ref, x2_ref, x3_ref, cidx_ref):
    cidx_ref[...] = ((x0_ref[...] << 9) | (x1_ref[...] << 6) |
                     (x2_ref[...] << 3) | x3_ref[...])


def _gather_body(cidx_hbm, t_hbm, out_hbm, cidx_v, rows_v,
                 sem_i, sem_g, sem_w0, sem_w1):
    wid = lax.axis_index("s") * NC + lax.axis_index("c")
    idx_base = wid * IDX_PER_W
    out_base = wid * ROWS_PER_W
    sem_w = (sem_w0, sem_w1)

    def stage_idx(c, slot):
        return pltpu.async_copy(
            cidx_hbm.at[pl.ds(idx_base + c * IDX_ROWS, IDX_ROWS)],
            cidx_v.at[slot], sem_i)

    def step(c, b, *, wait_idx, drain_write, prefetch):
        if wait_idx:
            pltpu.make_async_copy(
                cidx_hbm.at[pl.ds(0, IDX_ROWS)], cidx_v.at[b], sem_i).wait()
        if drain_write:
            pltpu.make_async_copy(
                out_hbm.at[pl.ds(0, CHUNK)], rows_v.at[b], sem_w[b]).wait()
        gs = [pltpu.async_copy(
                  t_hbm.at[cidx_v.at[b, j]],
                  rows_v.at[b, pl.ds(j * 128, 128)], sem_g)
              for j in range(IDX_ROWS)]
        if prefetch is not None:
            stage_idx(prefetch, 1 - b)
        for g in gs:
            g.wait()
        pltpu.async_copy(
            rows_v.at[b],
            out_hbm.at[pl.ds(out_base + c * CHUNK, CHUNK)], sem_w[b])

    stage_idx(0, 0).wait()
    step(0, 0, wait_idx=False, drain_write=False, prefetch=1)
    step(1, 1, wait_idx=True, drain_write=False, prefetch=2)

    def pair(i, carry):
        c0 = 2 * i
        step(c0, 0, wait_idx=True, drain_write=True, prefetch=c0 + 1)
        step(c0 + 1, 1, wait_idx=True, drain_write=True, prefetch=c0 + 2)
        return carry

    lax.fori_loop(1, NCHUNK // 2 - 1, pair, 0)

    step(NCHUNK - 2, 0, wait_idx=True, drain_write=True,
         prefetch=NCHUNK - 1)
    step(NCHUNK - 1, 1, wait_idx=True, drain_write=True, prefetch=None)
    for b in (0, 1):
        pltpu.make_async_copy(
            out_hbm.at[pl.ds(0, CHUNK)], rows_v.at[b], sem_w[b]).wait()


@jax.jit
def _run(x0, x1, x2, x3, wm, wd, ww, wh):
    table = pl.pallas_call(
        _build_table_body,
        out_shape=jax.ShapeDtypeStruct((4096, D_OUT), jnp.float32),
    )(wm, wd, ww, wh)

    cidx = pl.pallas_call(
        _fuse_idx_body,
        out_shape=jax.ShapeDtypeStruct((IDX_TOTAL, 128), jnp.int32),
    )(x0, x1, x2, x3)

    mesh = plsc.VectorSubcoreMesh(
        core_axis_name="c", subcore_axis_name="s",
        num_cores=NC, num_subcores=NS)
    gather = pl.kernel(
        _gather_body,
        out_type=jax.ShapeDtypeStruct((TOTAL, D_OUT), jnp.float32),
        mesh=mesh,
        scratch_types=[
            pltpu.VMEM((2, IDX_ROWS, 128), jnp.int32),
            pltpu.VMEM((2, CHUNK, D_OUT), jnp.float32),
            pltpu.SemaphoreType.DMA,
            pltpu.SemaphoreType.DMA,
            pltpu.SemaphoreType.DMA,
            pltpu.SemaphoreType.DMA,
        ],
    )
    return gather(cidx, table)


def kernel(x_time, W_month, W_day, W_weekday, W_hour):
    xt = x_time.astype(jnp.int32)
    planes = [xt[:, :, f].reshape(IDX_TOTAL, 128) for f in range(4)]
    out = _run(*planes, W_month, W_day, W_weekday, W_hour)
    return out.reshape(BATCH, SEQ, D_OUT)


# speedup vs baseline: 44.9920x; 1.7203x over previous
import jax
import jax.numpy as jnp
from jax import lax
from jax.experimental import pallas as pl
from jax.experimental.pallas import tpu as pltpu
from jax.experimental.pallas import tpu_sc as plsc

D_MODEL = 32
D_OUT = 4 * D_MODEL
BATCH = 4096
SEQ = 200
TOTAL = BATCH * SEQ
NC, NS = 2, 16
NW = NC * NS
ROWS_PER_W = TOTAL // NW
CHUNK = 256
NCHUNK = ROWS_PER_W // CHUNK
IDX_ROWS = CHUNK // 128
IDX_TOTAL = TOTAL // 128
IDX_PER_W = IDX_TOTAL // NW


def _prep_body(wm_ref, wd_ref, ww_ref, wh_ref,
               x0_ref, x1_ref, x2_ref, x3_ref, t_ref, cidx_ref):
    i = lax.broadcasted_iota(jnp.int32, (4096, 1), 0)
    parts = []
    for shift, rows, w_ref in ((9, 8, wm_ref), (6, 8, wd_ref),
                               (3, 7, ww_ref), (0, 8, wh_ref)):
        sub = (i >> shift) & 7
        acc = jnp.broadcast_to(w_ref[0:1, :], (4096, D_MODEL))
        for k in range(1, rows):
            acc = jnp.where(sub == k, w_ref[k:k + 1, :], acc)
        parts.append(acc)
    t_ref[...] = jnp.concatenate(parts, axis=1)
    cidx_ref[...] = ((x0_ref[...] << 9) | (x1_ref[...] << 6) |
                     (x2_ref[...] << 3) | x3_ref[...])


def _gather_body(cidx_hbm, t_hbm, out_hbm, t_sh, cidx_v, rows_v,
                 sem_i, sem_g, sem_w0, sem_w1):
    sid = lax.axis_index("s")
    wid = sid * NC + lax.axis_index("c")
    idx_base = wid * IDX_PER_W
    out_base = wid * ROWS_PER_W
    sem_w = (sem_w0, sem_w1)

    t_rows = 4096 // NS
    pltpu.sync_copy(t_hbm.at[pl.ds(sid * t_rows, t_rows)],
                    t_sh.at[pl.ds(sid * t_rows, t_rows)])
    plsc.subcore_barrier()

    def stage_idx(c, slot):
        return pltpu.async_copy(
            cidx_hbm.at[pl.ds(idx_base + c * IDX_ROWS, IDX_ROWS)],
            cidx_v.at[slot], sem_i)

    def step(c, b, *, wait_idx, drain_write, prefetch):
        if wait_idx:
            pltpu.make_async_copy(
                cidx_hbm.at[pl.ds(0, IDX_ROWS)], cidx_v.at[b], sem_i).wait()
        if drain_write:
            pltpu.make_async_copy(
                out_hbm.at[pl.ds(0, CHUNK)], rows_v.at[b], sem_w[b]).wait()
        gs = [pltpu.async_copy(
                  t_sh.at[cidx_v.at[b, j]],
                  rows_v.at[b, pl.ds(j * 128, 128)], sem_g)
              for j in range(IDX_ROWS)]
        if prefetch is not None:
            stage_idx(prefetch, 1 - b)
        for g in gs:
            g.wait()
        pltpu.async_copy(
            rows_v.at[b],
            out_hbm.at[pl.ds(out_base + c * CHUNK, CHUNK)], sem_w[b])

    stage_idx(0, 0).wait()
    step(0, 0, wait_idx=False, drain_write=False, prefetch=1)
    step(1, 1, wait_idx=True, drain_write=False, prefetch=2)

    def pair(i, carry):
        c0 = 2 * i
        step(c0, 0, wait_idx=True, drain_write=True, prefetch=c0 + 1)
        step(c0 + 1, 1, wait_idx=True, drain_write=True, prefetch=c0 + 2)
        return carry

    lax.fori_loop(1, NCHUNK // 2 - 1, pair, 0)

    step(NCHUNK - 2, 0, wait_idx=True, drain_write=True,
         prefetch=NCHUNK - 1)
    step(NCHUNK - 1, 1, wait_idx=True, drain_write=True, prefetch=None)
    for b in (0, 1):
        pltpu.make_async_copy(
            out_hbm.at[pl.ds(0, CHUNK)], rows_v.at[b], sem_w[b]).wait()


@jax.jit
def _run(x0, x1, x2, x3, wm, wd, ww, wh):
    table, cidx = pl.pallas_call(
        _prep_body,
        out_shape=[jax.ShapeDtypeStruct((4096, D_OUT), jnp.float32),
                   jax.ShapeDtypeStruct((IDX_TOTAL, 128), jnp.int32)],
    )(wm, wd, ww, wh, x0, x1, x2, x3)

    mesh = plsc.VectorSubcoreMesh(
        core_axis_name="c", subcore_axis_name="s",
        num_cores=NC, num_subcores=NS)
    gather = pl.kernel(
        _gather_body,
        out_type=jax.ShapeDtypeStruct((TOTAL, D_OUT), jnp.float32),
        mesh=mesh,
        scratch_types=[
            pltpu.VMEM_SHARED((4096, D_OUT), jnp.float32),
            pltpu.VMEM((2, IDX_ROWS, 128), jnp.int32),
            pltpu.VMEM((2, CHUNK, D_OUT), jnp.float32),
            pltpu.SemaphoreType.DMA,
            pltpu.SemaphoreType.DMA,
            pltpu.SemaphoreType.DMA,
            pltpu.SemaphoreType.DMA,
        ],
    )
    return gather(cidx, table)


def kernel(x_time, W_month, W_day, W_weekday, W_hour):
    xt = x_time.astype(jnp.int32)
    planes = [xt[:, :, f].reshape(IDX_TOTAL, 128) for f in range(4)]
    out = _run(*planes, W_month, W_day, W_weekday, W_hour)
    return out.reshape(BATCH, SEQ, D_OUT)


# speedup vs baseline: 52.9063x; 1.1759x over previous
import jax
import jax.numpy as jnp
from jax import lax
from jax.experimental import pallas as pl
from jax.experimental.pallas import tpu as pltpu
from jax.experimental.pallas import tpu_sc as plsc

D_MODEL = 32
D_OUT = 4 * D_MODEL
BATCH = 4096
SEQ = 200
TOTAL = BATCH * SEQ
NC, NS, L = 2, 16, 16
NW = NC * NS
ROWS_PER_W = TOTAL // NW
CHUNK = 256
NCHUNK = ROWS_PER_W // CHUNK
IDX_ROWS = CHUNK // 128
IDX_TOTAL = TOTAL // 128
IDX_PER_W = IDX_TOTAL // NW


def _table_body(wm_ref, wd_ref, ww_ref, wh_ref, t_ref):
    i = lax.broadcasted_iota(jnp.int32, (4096, 1), 0)
    parts = []
    for shift, rows, w_ref in ((9, 8, wm_ref), (6, 8, wd_ref),
                               (3, 7, ww_ref), (0, 8, wh_ref)):
        sub = (i >> shift) & 7
        acc = jnp.broadcast_to(w_ref[0:1, :], (4096, D_MODEL))
        for k in range(1, rows):
            acc = jnp.where(sub == k, w_ref[k:k + 1, :], acc)
        parts.append(acc)
    t_ref[...] = jnp.concatenate(parts, axis=1)


def _gather_body(xq_hbm, t_hbm, out_hbm, t_sh, idx_v, cidx_v, rows_v,
                 sem_i, sem_g0, sem_g1, sem_w0, sem_w1):
    sid = lax.axis_index("s")
    wid = sid * NC + lax.axis_index("c")
    idx_base = wid * IDX_PER_W
    out_base = wid * ROWS_PER_W
    sem_g = (sem_g0, sem_g1)
    sem_w = (sem_w0, sem_w1)

    t_rows = 4096 // NS
    pltpu.sync_copy(t_hbm.at[pl.ds(sid * t_rows, t_rows)],
                    t_sh.at[pl.ds(sid * t_rows, t_rows)])
    plsc.subcore_barrier()

    def stage(c, slot):
        return pltpu.async_copy(
            xq_hbm.at[pl.ds(idx_base + c * IDX_ROWS, IDX_ROWS)],
            idx_v.at[slot], sem_i)

    def combine(slot):
        for j in range(IDX_ROWS):
            for p in range(0, 128, L):
                s = pl.ds(p, L)
                cidx_v[slot, j, s] = (
                    (idx_v[slot, j, 0, s] << 9) |
                    (idx_v[slot, j, 1, s] << 6) |
                    (idx_v[slot, j, 2, s] << 3) |
                    idx_v[slot, j, 3, s])

    def fire_gather(slot):
        for j in range(IDX_ROWS):
            pltpu.async_copy(
                t_sh.at[cidx_v.at[slot, j]],
                rows_v.at[slot, pl.ds(j * 128, 128)], sem_g[slot])

    def wait_gather(slot):
        pltpu.make_async_copy(
            out_hbm.at[pl.ds(0, CHUNK)], rows_v.at[slot],
            sem_g[slot]).wait()

    def fire_write(c, slot):
        pltpu.async_copy(
            rows_v.at[slot],
            out_hbm.at[pl.ds(out_base + c * CHUNK, CHUNK)], sem_w[slot])

    def drain_write(slot):
        pltpu.make_async_copy(
            out_hbm.at[pl.ds(0, CHUNK)], rows_v.at[slot],
            sem_w[slot]).wait()

    def wait_stage(slot):
        pltpu.make_async_copy(
            xq_hbm.at[pl.ds(0, IDX_ROWS)], idx_v.at[slot], sem_i).wait()

    def step(c, b, *, drain, nxt_gather, nxt_stage, last):
        if not last:
            combine(1 - b)
        if nxt_stage:
            stage(c + 2, b)
        if drain:
            drain_write(1 - b)
        if nxt_gather:
            fire_gather(1 - b)
        wait_gather(b)
        fire_write(c, b)
        if nxt_stage:
            wait_stage(b)

    stage(0, 0).wait()
    combine(0)
    fire_gather(0)
    stage(1, 1).wait()

    step(0, 0, drain=False, nxt_gather=True, nxt_stage=True, last=False)
    step(1, 1, drain=True, nxt_gather=True, nxt_stage=True, last=False)

    def pair(i, carry):
        c0 = 2 * i
        step(c0, 0, drain=True, nxt_gather=True, nxt_stage=True,
             last=False)
        step(c0 + 1, 1, drain=True, nxt_gather=True, nxt_stage=True,
             last=False)
        return carry

    lax.fori_loop(1, NCHUNK // 2 - 1, pair, 0)

    step(NCHUNK - 2, 0, drain=True, nxt_gather=True, nxt_stage=False,
         last=False)
    step(NCHUNK - 1, 1, drain=True, nxt_gather=False, nxt_stage=False,
         last=True)
    drain_write(1)


@jax.jit
def _run(xq, wm, wd, ww, wh):
    table = pl.pallas_call(
        _table_body,
        out_shape=jax.ShapeDtypeStruct((4096, D_OUT), jnp.float32),
    )(wm, wd, ww, wh)

    mesh = plsc.VectorSubcoreMesh(
        core_axis_name="c", subcore_axis_name="s",
        num_cores=NC, num_subcores=NS)
    gather = pl.kernel(
        _gather_body,
        out_type=jax.ShapeDtypeStruct((TOTAL, D_OUT), jnp.float32),
        mesh=mesh,
        scratch_types=[
            pltpu.VMEM_SHARED((4096, D_OUT), jnp.float32),
            pltpu.VMEM((2, IDX_ROWS, 4, 128), jnp.int32),
            pltpu.VMEM((2, IDX_ROWS, 128), jnp.int32),
            pltpu.VMEM((2, CHUNK, D_OUT), jnp.float32),
            pltpu.SemaphoreType.DMA,
            pltpu.SemaphoreType.DMA,
            pltpu.SemaphoreType.DMA,
            pltpu.SemaphoreType.DMA,
            pltpu.SemaphoreType.DMA,
        ],
    )
    return gather(xq, table)


def kernel(x_time, W_month, W_day, W_weekday, W_hour):
    xq = x_time.astype(jnp.int32).reshape(IDX_TOTAL, 128, 4).swapaxes(1, 2)
    out = _run(xq, W_month, W_day, W_weekday, W_hour)
    return out.reshape(BATCH, SEQ, D_OUT)
